# baseline (speedup 1.0000x reference)
import functools

import jax
import jax.numpy as jnp
from jax import lax
from jax.experimental import pallas as pl
from jax.experimental.pallas import tpu as pltpu
from jax.experimental.pallas import tpu_sc as plsc

_B, _S, _C, _K = 32, 1024, 768, 128
_NCLS = 200


_CH = 128


def _select_body(logits_ref, idx_ref, key_scr, rank_scr):
    l = logits_ref[0]
    m = jnp.max(l, axis=-1, keepdims=True)
    s = jnp.sum(jnp.exp(l - m), axis=-1)
    key_scr[:] = 1.0 / s
    key = key_scr[:]
    b = pl.program_id(0)

    def rank_chunk(ci, carry):
        kc = key_scr[pl.ds(ci * _CH, _CH)]
        ii = ci * _CH + lax.broadcasted_iota(jnp.int32, (_CH, _S), 0)
        jj = lax.broadcasted_iota(jnp.int32, (_CH, _S), 1)
        beats = (key[None, :] > kc[:, None]) | (
            (key[None, :] == kc[:, None]) & (jj < ii)
        )
        rank_scr[pl.ds(ci * _CH, _CH)] = jnp.sum(beats.astype(jnp.int32), axis=-1)
        return carry

    lax.fori_loop(0, _S // _CH, rank_chunk, 0)
    rank = rank_scr[:]
    rr = lax.broadcasted_iota(jnp.int32, (_K, _S), 0)
    hit = rank[None, :] == rr
    jglob = b * _S + lax.broadcasted_iota(jnp.int32, (_K, _S), 1)
    idx_ref[0, 0, :] = jnp.sum(jnp.where(hit, jglob, 0), axis=-1)


def _select_indices(logits):
    idx3 = pl.pallas_call(
        _select_body,
        grid=(_B,),
        in_specs=[pl.BlockSpec((1, _S, _NCLS), lambda b: (b, 0, 0))],
        out_specs=pl.BlockSpec((1, 1, _K), lambda b: (b, 0, 0)),
        out_shape=jax.ShapeDtypeStruct((_B, 1, _K), jnp.int32),
        scratch_shapes=[
            pltpu.VMEM((_S,), jnp.float32),
            pltpu.VMEM((_S,), jnp.int32),
        ],
    )(logits)
    return idx3.reshape(_B * _K)


def _make_sc_gather():
    info = plsc.get_sparse_core_info()
    nw = info.num_cores * info.num_subcores
    rows_per_w = (_B * _K) // nw
    mesh = plsc.VectorSubcoreMesh(core_axis_name="c", subcore_axis_name="s")

    @functools.partial(
        pl.kernel,
        mesh=mesh,
        out_type=jax.ShapeDtypeStruct((_B * _K, _C), jnp.float32),
        scratch_types=[
            pltpu.VMEM((rows_per_w,), jnp.int32),
            pltpu.VMEM((rows_per_w, _C), jnp.float32),
            pltpu.SemaphoreType.DMA,
        ],
    )
    def gather(table_hbm, idx_hbm, out_hbm, idx_v, rows_v, sem):
        wid = lax.axis_index("s") * info.num_cores + lax.axis_index("c")
        base = wid * rows_per_w
        pltpu.sync_copy(idx_hbm.at[pl.ds(base, rows_per_w)], idx_v)
        pltpu.async_copy(table_hbm.at[idx_v], rows_v, sem).wait()
        pltpu.sync_copy(rows_v, out_hbm.at[pl.ds(base, rows_per_w)])

    return gather


def kernel(feat, logits):
    flat_idx = _select_indices(logits)
    gathered = _make_sc_gather()(feat.reshape(_B * _S, _C), flat_idx)
    return gathered.reshape(_B, _K, _C)
